# Initial kernel scaffold; baseline (speedup 1.0000x reference)
import jax
import jax.numpy as jnp
from jax.experimental import pallas as pl


def kernel(x, y, q_w, q_dw_w, kv_w, kv_dw_w, proj_w, gate_w1, gate_b1, gate_w2, gate_b2, temperature):
    raise NotImplementedError("write your pallas kernel here")


# speedup vs baseline: 1.6429x; 1.6429x over previous
import jax
import jax.numpy as jnp
from jax import lax
from jax.experimental import pallas as pl
from jax.experimental.pallas import tpu as pltpu

C = 192
HEADS = 6
CH = 32
HH = 384
WW = 384
TH_A = 8
TH_C = 16
NSTEPS_A = HH // TH_A
NSTEPS_C = HH // TH_C
NPIX = HH * WW
NEG_INF = float("-inf")
BF = jnp.bfloat16
F32 = jnp.float32


def _dwconv_emul(p_full, w9, th):
    pf = (th + 2) * WW
    m = th * WW
    col = lax.rem(lax.broadcasted_iota(jnp.int32, (1, pf), 1), WW)
    pb = p_full.astype(BF).astype(F32)
    zero1 = jnp.zeros((C, 1), F32)
    left = jnp.where(col == 0, 0.0,
                     jnp.concatenate([zero1, pb[:, :pf - 1]], axis=1))
    right = jnp.where(col == WW - 1, 0.0,
                      jnp.concatenate([pb[:, 1:], zero1], axis=1))
    srcs = (left, pb, right)
    acc = None
    for dy in range(3):
        for dx in range(3):
            t = dy * 3 + dx
            term = w9[:, t:t + 1] * srcs[dx][:, dy * WW:dy * WW + m]
            acc = term if acc is None else acc + term
    return acc


def _qk_tiles(xc, xb_ref, yc_ref, yb_ref, qw_ref, qdw_ref, kw_ref, kdw_ref):
    xfull = jnp.concatenate([xb_ref[0, 0], xc, xb_ref[0, 1]], axis=1)
    qp = jnp.dot(qw_ref[...], xfull.astype(BF),
                 preferred_element_type=F32)
    dwq = _dwconv_emul(qp, qdw_ref[...], TH_A)
    yfull = jnp.concatenate([yb_ref[0, 0], yc_ref[...], yb_ref[0, 1]],
                            axis=1)
    kp = jnp.dot(kw_ref[...], yfull.astype(BF),
                 preferred_element_type=F32)
    dwk = _dwconv_emul(kp, kdw_ref[...], TH_A)
    return dwq, dwk


def _pass_a_body(xc_ref, xb_ref, yc_ref, yb_ref, qw_ref, qdw_ref, kw_ref,
                 kdw_ref, w1_ref, b1_ref, w2_ref, b2_ref, temp_ref,
                 attn_ref, dk_ref, qsq_ref, ksq_ref, gs_ref):
    p = pl.program_id(0)
    i = pl.program_id(1)

    @pl.when((p == 0) & (i == 0))
    def _init():
        qsq_ref[...] = jnp.zeros_like(qsq_ref)
        ksq_ref[...] = jnp.zeros_like(ksq_ref)
        gs_ref[0, 0] = 0.0

    xc = xc_ref[...]
    dwq, dwk = _qk_tiles(xc, xb_ref, yc_ref, yb_ref, qw_ref, qdw_ref,
                         kw_ref, kdw_ref)

    @pl.when(p == 0)
    def _phase0():
        qsq_ref[...] += jnp.sum(dwq * dwq, axis=1, keepdims=True)
        ksq_ref[...] += jnp.sum(dwk * dwk, axis=1, keepdims=True)
        z1 = jnp.maximum(jnp.dot(w1_ref[...], xc.astype(BF),
                                 preferred_element_type=F32)
                         + b1_ref[...], 0.0)
        z2 = jax.nn.sigmoid(jnp.dot(w2_ref[...], z1.astype(BF),
                                    preferred_element_type=F32)
                            + b2_ref[...])
        z2 = jnp.where(jnp.isnan(z2), 0.0, z2)
        gs_ref[0, 0] += jnp.sum(z2)

        @pl.when(i == NSTEPS_A - 1)
        def _finalize_norms():
            qsq_ref[...] = jnp.maximum(jnp.sqrt(qsq_ref[...]), 1e-12)
            ksq_ref[...] = jnp.maximum(jnp.sqrt(ksq_ref[...]), 1e-12)
            gsv = gs_ref[0, 0]
            dkv = jnp.floor(CH * gsv / NPIX).astype(jnp.int32)
            dk_ref[0, 0] = jnp.maximum(dkv, 1)

    @pl.when(p == 1)
    def _phase1():
        @pl.when(i == 0)
        def _init_attn():
            attn_ref[...] = jnp.zeros_like(attn_ref)

        qn = (dwq / qsq_ref[...]).astype(BF)
        kn = (dwk / ksq_ref[...]).astype(BF)
        for h in range(HEADS):
            s_h = lax.dot_general(qn[h * CH:(h + 1) * CH],
                                  kn[h * CH:(h + 1) * CH],
                                  (((1,), (1,)), ((), ())),
                                  preferred_element_type=F32)
            attn_ref[h] += s_h

        @pl.when(i == NSTEPS_A - 1)
        def _finalize_attn():
            for h in range(HEADS):
                attn_ref[h] = attn_ref[h] * temp_ref[h, 0]


def _middle_body(a_ref, dk_ref, out_ref):
    a = a_ref[...].reshape(C, CH)
    dkv = dk_ref[0, 0]
    col = lax.broadcasted_iota(jnp.int32, (C, CH), 1)
    rank = jnp.zeros((C, CH), jnp.int32)
    for e in range(CH):
        ae = a[:, e:e + 1]
        pred = (ae > a) | ((ae == a) & (e < col))
        rank = rank + pred.astype(jnp.int32)
    mask = rank < dkv
    am = jnp.where(mask, a, NEG_INF)
    mx = jnp.max(am, axis=1, keepdims=True)
    ex = jnp.where(mask, jnp.exp(am - mx), 0.0)
    s = jnp.sum(ex, axis=1, keepdims=True)
    out_ref[...] = (ex / s).reshape(HEADS, CH, CH)


def _pass_c_body(yc_ref, yb_ref, vw_ref, vdw_ref, proj_ref, attn_ref,
                 out_ref):
    yfull = jnp.concatenate([yb_ref[0, 0], yc_ref[...], yb_ref[0, 1]],
                            axis=1)
    vp = jnp.dot(vw_ref[...], yfull.astype(BF), preferred_element_type=F32)
    dwv = _dwconv_emul_c(vp, vdw_ref[...])

    mm = jnp.concatenate(
        [jnp.dot(proj_ref[:, h * CH:(h + 1) * CH], attn_ref[h],
                 preferred_element_type=F32,
                 precision=lax.Precision.HIGHEST)
         for h in range(HEADS)], axis=1)
    out_ref[...] = jnp.dot(mm.astype(BF), dwv.astype(BF),
                           preferred_element_type=F32)


def _dwconv_emul_c(p_full, w9):
    return _dwconv_emul(p_full, w9, TH_C)


def _boundary_rows(x3, th):
    prev = x3[:, th - 1:HH - 1:th, :]
    nxt = x3[:, th:HH:th, :]
    zero = jnp.zeros((C, 1, WW), x3.dtype)
    prev_all = jnp.concatenate([zero, prev], axis=1).transpose(1, 0, 2)
    next_all = jnp.concatenate([nxt, zero], axis=1).transpose(1, 0, 2)
    return jnp.stack([prev_all, next_all], axis=1)


def kernel(x, y, q_w, q_dw_w, kv_w, kv_dw_w, proj_w, gate_w1, gate_b1,
           gate_w2, gate_b2, temperature):
    x3 = x.reshape(C, HH, WW)
    y3 = y.reshape(C, HH, WW)
    x2 = x.reshape(C, NPIX)
    y2 = y.reshape(C, NPIX)
    xb = _boundary_rows(x3, TH_A)
    yb = _boundary_rows(y3, TH_A)
    yb_c = _boundary_rows(y3, TH_C)
    k_w = kv_w[:C].astype(BF)
    v_w = kv_w[C:].astype(BF)
    qw_b = q_w.astype(BF)
    qdw = q_dw_w.reshape(C, 9)
    kdw = kv_dw_w[:C].reshape(C, 9)
    vdw = kv_dw_w[C:].reshape(C, 9)
    w1_b = gate_w1.astype(BF)
    w2_b = gate_w2.astype(BF)
    b1 = gate_b1.reshape(C // 2, 1)
    b2 = gate_b2.reshape(1, 1)
    temp = temperature.reshape(HEADS, 1)

    full = lambda shape: pl.BlockSpec(shape, lambda p, i: (0,) * len(shape))
    attn_s, dk = pl.pallas_call(
        _pass_a_body,
        grid=(2, NSTEPS_A),
        in_specs=[
            pl.BlockSpec((C, TH_A * WW), lambda p, i: (0, i)),
            pl.BlockSpec((1, 2, C, WW), lambda p, i: (i, 0, 0, 0)),
            pl.BlockSpec((C, TH_A * WW), lambda p, i: (0, i)),
            pl.BlockSpec((1, 2, C, WW), lambda p, i: (i, 0, 0, 0)),
            full((C, C)),
            full((C, 9)),
            full((C, C)),
            full((C, 9)),
            full((C // 2, C)),
            full((C // 2, 1)),
            full((1, C // 2)),
            full((1, 1)),
            pl.BlockSpec(memory_space=pltpu.SMEM),
        ],
        out_specs=[
            pl.BlockSpec((HEADS, CH, CH), lambda p, i: (0, 0, 0)),
            pl.BlockSpec(memory_space=pltpu.SMEM),
        ],
        out_shape=[
            jax.ShapeDtypeStruct((HEADS, CH, CH), jnp.float32),
            jax.ShapeDtypeStruct((1, 1), jnp.int32),
        ],
        scratch_shapes=[
            pltpu.VMEM((C, 1), jnp.float32),
            pltpu.VMEM((C, 1), jnp.float32),
            pltpu.SMEM((1, 1), jnp.float32),
        ],
        compiler_params=pltpu.CompilerParams(
            dimension_semantics=("arbitrary", "arbitrary")),
    )(x2, xb, y2, yb, qw_b, qdw, k_w, kdw, w1_b, b1, w2_b, b2, temp)

    attn = pl.pallas_call(
        _middle_body,
        in_specs=[
            pl.BlockSpec((HEADS, CH, CH), lambda: (0, 0, 0)),
            pl.BlockSpec(memory_space=pltpu.SMEM),
        ],
        out_specs=pl.BlockSpec((HEADS, CH, CH), lambda: (0, 0, 0)),
        out_shape=jax.ShapeDtypeStruct((HEADS, CH, CH), jnp.float32),
    )(attn_s, dk)

    out = pl.pallas_call(
        _pass_c_body,
        grid=(NSTEPS_C,),
        in_specs=[
            pl.BlockSpec((C, TH_C * WW), lambda i: (0, i)),
            pl.BlockSpec((1, 2, C, WW), lambda i: (i, 0, 0, 0)),
            pl.BlockSpec((C, C), lambda i: (0, 0)),
            pl.BlockSpec((C, 9), lambda i: (0, 0)),
            pl.BlockSpec((C, C), lambda i: (0, 0)),
            pl.BlockSpec((HEADS, CH, CH), lambda i: (0, 0, 0)),
        ],
        out_specs=pl.BlockSpec((C, TH_C * WW), lambda i: (0, i)),
        out_shape=jax.ShapeDtypeStruct((C, NPIX), jnp.float32),
        compiler_params=pltpu.CompilerParams(
            dimension_semantics=("arbitrary",)),
    )(y2, yb_c, v_w, vdw, proj_w, attn)

    return out.reshape(1, C, HH, WW)


# speedup vs baseline: 1.7433x; 1.0611x over previous
import jax
import jax.numpy as jnp
from jax import lax
from jax.experimental import pallas as pl
from jax.experimental.pallas import tpu as pltpu

C = 192
HEADS = 6
CH = 32
HH = 384
WW = 384
TH_A = 12
TH_C = 16
NSTEPS_A = HH // TH_A
NSTEPS_C = HH // TH_C
NPIX = HH * WW
NEG_INF = float("-inf")
BF = jnp.bfloat16
F32 = jnp.float32


def _dwconv_emul(p_full, w9, th):
    pf = (th + 2) * WW
    m = th * WW
    col = lax.rem(lax.broadcasted_iota(jnp.int32, (1, pf), 1), WW)
    pb = p_full.astype(BF).astype(F32)
    zero1 = jnp.zeros((C, 1), F32)
    left = jnp.where(col == 0, 0.0,
                     jnp.concatenate([zero1, pb[:, :pf - 1]], axis=1))
    right = jnp.where(col == WW - 1, 0.0,
                      jnp.concatenate([pb[:, 1:], zero1], axis=1))
    srcs = (left, pb, right)
    acc = None
    for dy in range(3):
        for dx in range(3):
            t = dy * 3 + dx
            term = w9[:, t:t + 1] * srcs[dx][:, dy * WW:dy * WW + m]
            acc = term if acc is None else acc + term
    return acc


def _qk_tiles(xc, xb_ref, yc_ref, yb_ref, qw_ref, qdw_ref, kw_ref, kdw_ref):
    xfull = jnp.concatenate([xb_ref[0, 0], xc, xb_ref[0, 1]], axis=1)
    qp = jnp.dot(qw_ref[...], xfull.astype(BF),
                 preferred_element_type=F32)
    dwq = _dwconv_emul(qp, qdw_ref[...], TH_A)
    yfull = jnp.concatenate([yb_ref[0, 0], yc_ref[...], yb_ref[0, 1]],
                            axis=1)
    kp = jnp.dot(kw_ref[...], yfull.astype(BF),
                 preferred_element_type=F32)
    dwk = _dwconv_emul(kp, kdw_ref[...], TH_A)
    return dwq, dwk


def _pass_a_body(xc_ref, xb_ref, yc_ref, yb_ref, qw_ref, qdw_ref, kw_ref,
                 kdw_ref, w1_ref, b1_ref, w2_ref, b2_ref, temp_ref,
                 attn_ref, dk_ref, qsq_ref, ksq_ref, gs_ref):
    p = pl.program_id(0)
    i = pl.program_id(1)

    @pl.when((p == 0) & (i == 0))
    def _init():
        qsq_ref[...] = jnp.zeros_like(qsq_ref)
        ksq_ref[...] = jnp.zeros_like(ksq_ref)
        gs_ref[0, 0] = 0.0

    xc = xc_ref[...]
    dwq, dwk = _qk_tiles(xc, xb_ref, yc_ref, yb_ref, qw_ref, qdw_ref,
                         kw_ref, kdw_ref)

    @pl.when(p == 0)
    def _phase0():
        qsq_ref[...] += jnp.sum(dwq * dwq, axis=1, keepdims=True)
        ksq_ref[...] += jnp.sum(dwk * dwk, axis=1, keepdims=True)
        z1 = jnp.maximum(jnp.dot(w1_ref[...], xc.astype(BF),
                                 preferred_element_type=F32)
                         + b1_ref[...], 0.0)
        z2 = jax.nn.sigmoid(jnp.dot(w2_ref[...], z1.astype(BF),
                                    preferred_element_type=F32)
                            + b2_ref[...])
        z2 = jnp.where(jnp.isnan(z2), 0.0, z2)
        gs_ref[0, 0] += jnp.sum(z2)

        @pl.when(i == NSTEPS_A - 1)
        def _finalize_norms():
            qsq_ref[...] = jnp.maximum(jnp.sqrt(qsq_ref[...]), 1e-12)
            ksq_ref[...] = jnp.maximum(jnp.sqrt(ksq_ref[...]), 1e-12)
            gsv = gs_ref[0, 0]
            dkv = jnp.floor(CH * gsv / NPIX).astype(jnp.int32)
            dk_ref[0, 0] = jnp.maximum(dkv, 1)

    @pl.when(p == 1)
    def _phase1():
        @pl.when(i == 0)
        def _init_attn():
            attn_ref[...] = jnp.zeros_like(attn_ref)

        qn = (dwq / qsq_ref[...]).astype(BF)
        kn = (dwk / ksq_ref[...]).astype(BF)
        for h in range(HEADS):
            s_h = lax.dot_general(qn[h * CH:(h + 1) * CH],
                                  kn[h * CH:(h + 1) * CH],
                                  (((1,), (1,)), ((), ())),
                                  preferred_element_type=F32)
            attn_ref[h] += s_h

        @pl.when(i == NSTEPS_A - 1)
        def _finalize_attn():
            for h in range(HEADS):
                attn_ref[h] = attn_ref[h] * temp_ref[h, 0]


def _middle_body(a_ref, dk_ref, out_ref):
    a = a_ref[...].reshape(C, CH)
    dkv = dk_ref[0, 0]
    col = lax.broadcasted_iota(jnp.int32, (C, CH), 1)
    rank = jnp.zeros((C, CH), jnp.int32)
    for e in range(CH):
        ae = a[:, e:e + 1]
        pred = (ae > a) | ((ae == a) & (e < col))
        rank = rank + pred.astype(jnp.int32)
    mask = rank < dkv
    am = jnp.where(mask, a, NEG_INF)
    mx = jnp.max(am, axis=1, keepdims=True)
    ex = jnp.where(mask, jnp.exp(am - mx), 0.0)
    s = jnp.sum(ex, axis=1, keepdims=True)
    out_ref[...] = (ex / s).reshape(HEADS, CH, CH)


def _pass_c_body(yc_ref, yb_ref, vw_ref, vdw_ref, proj_ref, attn_ref,
                 out_ref):
    yfull = jnp.concatenate([yb_ref[0, 0], yc_ref[...], yb_ref[0, 1]],
                            axis=1)
    vp = jnp.dot(vw_ref[...], yfull.astype(BF), preferred_element_type=F32)
    dwv = _dwconv_emul_c(vp, vdw_ref[...])

    mm = jnp.concatenate(
        [jnp.dot(proj_ref[:, h * CH:(h + 1) * CH], attn_ref[h],
                 preferred_element_type=F32,
                 precision=lax.Precision.HIGHEST)
         for h in range(HEADS)], axis=1)
    out_ref[...] = jnp.dot(mm.astype(BF), dwv.astype(BF),
                           preferred_element_type=F32)


def _dwconv_emul_c(p_full, w9):
    return _dwconv_emul(p_full, w9, TH_C)


def _boundary_rows(x3, th):
    prev = x3[:, th - 1:HH - 1:th, :]
    nxt = x3[:, th:HH:th, :]
    zero = jnp.zeros((C, 1, WW), x3.dtype)
    prev_all = jnp.concatenate([zero, prev], axis=1).transpose(1, 0, 2)
    next_all = jnp.concatenate([nxt, zero], axis=1).transpose(1, 0, 2)
    return jnp.stack([prev_all, next_all], axis=1)


def kernel(x, y, q_w, q_dw_w, kv_w, kv_dw_w, proj_w, gate_w1, gate_b1,
           gate_w2, gate_b2, temperature):
    x3 = x.reshape(C, HH, WW)
    y3 = y.reshape(C, HH, WW)
    x2 = x.reshape(C, NPIX)
    y2 = y.reshape(C, NPIX)
    xb = _boundary_rows(x3, TH_A)
    yb = _boundary_rows(y3, TH_A)
    yb_c = _boundary_rows(y3, TH_C)
    k_w = kv_w[:C].astype(BF)
    v_w = kv_w[C:].astype(BF)
    qw_b = q_w.astype(BF)
    qdw = q_dw_w.reshape(C, 9)
    kdw = kv_dw_w[:C].reshape(C, 9)
    vdw = kv_dw_w[C:].reshape(C, 9)
    w1_b = gate_w1.astype(BF)
    w2_b = gate_w2.astype(BF)
    b1 = gate_b1.reshape(C // 2, 1)
    b2 = gate_b2.reshape(1, 1)
    temp = temperature.reshape(HEADS, 1)

    full = lambda shape: pl.BlockSpec(shape, lambda p, i: (0,) * len(shape))
    attn_s, dk = pl.pallas_call(
        _pass_a_body,
        grid=(2, NSTEPS_A),
        in_specs=[
            pl.BlockSpec((C, TH_A * WW), lambda p, i: (0, i)),
            pl.BlockSpec((1, 2, C, WW), lambda p, i: (i, 0, 0, 0)),
            pl.BlockSpec((C, TH_A * WW), lambda p, i: (0, i)),
            pl.BlockSpec((1, 2, C, WW), lambda p, i: (i, 0, 0, 0)),
            full((C, C)),
            full((C, 9)),
            full((C, C)),
            full((C, 9)),
            full((C // 2, C)),
            full((C // 2, 1)),
            full((1, C // 2)),
            full((1, 1)),
            pl.BlockSpec(memory_space=pltpu.SMEM),
        ],
        out_specs=[
            pl.BlockSpec((HEADS, CH, CH), lambda p, i: (0, 0, 0)),
            pl.BlockSpec(memory_space=pltpu.SMEM),
        ],
        out_shape=[
            jax.ShapeDtypeStruct((HEADS, CH, CH), jnp.float32),
            jax.ShapeDtypeStruct((1, 1), jnp.int32),
        ],
        scratch_shapes=[
            pltpu.VMEM((C, 1), jnp.float32),
            pltpu.VMEM((C, 1), jnp.float32),
            pltpu.SMEM((1, 1), jnp.float32),
        ],
        compiler_params=pltpu.CompilerParams(
            dimension_semantics=("arbitrary", "arbitrary")),
    )(x2, xb, y2, yb, qw_b, qdw, k_w, kdw, w1_b, b1, w2_b, b2, temp)

    attn = pl.pallas_call(
        _middle_body,
        in_specs=[
            pl.BlockSpec((HEADS, CH, CH), lambda: (0, 0, 0)),
            pl.BlockSpec(memory_space=pltpu.SMEM),
        ],
        out_specs=pl.BlockSpec((HEADS, CH, CH), lambda: (0, 0, 0)),
        out_shape=jax.ShapeDtypeStruct((HEADS, CH, CH), jnp.float32),
    )(attn_s, dk)

    out = pl.pallas_call(
        _pass_c_body,
        grid=(NSTEPS_C,),
        in_specs=[
            pl.BlockSpec((C, TH_C * WW), lambda i: (0, i)),
            pl.BlockSpec((1, 2, C, WW), lambda i: (i, 0, 0, 0)),
            pl.BlockSpec((C, C), lambda i: (0, 0)),
            pl.BlockSpec((C, 9), lambda i: (0, 0)),
            pl.BlockSpec((C, C), lambda i: (0, 0)),
            pl.BlockSpec((HEADS, CH, CH), lambda i: (0, 0, 0)),
        ],
        out_specs=pl.BlockSpec((C, TH_C * WW), lambda i: (0, i)),
        out_shape=jax.ShapeDtypeStruct((C, NPIX), jnp.float32),
        compiler_params=pltpu.CompilerParams(
            dimension_semantics=("arbitrary",)),
    )(y2, yb_c, v_w, vdw, proj_w, attn)

    return out.reshape(1, C, HH, WW)


# speedup vs baseline: 1.7758x; 1.0186x over previous
import jax
import jax.numpy as jnp
from jax import lax
from jax.experimental import pallas as pl
from jax.experimental.pallas import tpu as pltpu

C = 192
HEADS = 6
CH = 32
HH = 384
WW = 384
TH_A = 12
TH_C = 24
NSTEPS_A = HH // TH_A
NSTEPS_C = HH // TH_C
NPIX = HH * WW
NEG_INF = float("-inf")
BF = jnp.bfloat16
F32 = jnp.float32


def _dwconv_emul(p_full, w9, th):
    pf = (th + 2) * WW
    m = th * WW
    col = lax.rem(lax.broadcasted_iota(jnp.int32, (1, pf), 1), WW)
    pb = p_full.astype(BF).astype(F32)
    zero1 = jnp.zeros((C, 1), F32)
    left = jnp.where(col == 0, 0.0,
                     jnp.concatenate([zero1, pb[:, :pf - 1]], axis=1))
    right = jnp.where(col == WW - 1, 0.0,
                      jnp.concatenate([pb[:, 1:], zero1], axis=1))
    srcs = (left, pb, right)
    acc = None
    for dy in range(3):
        for dx in range(3):
            t = dy * 3 + dx
            term = w9[:, t:t + 1] * srcs[dx][:, dy * WW:dy * WW + m]
            acc = term if acc is None else acc + term
    return acc


def _qk_tiles(xc, xb_ref, yc_ref, yb_ref, qw_ref, qdw_ref, kw_ref, kdw_ref):
    xfull = jnp.concatenate([xb_ref[0, 0], xc, xb_ref[0, 1]], axis=1)
    qp = jnp.dot(qw_ref[...], xfull.astype(BF),
                 preferred_element_type=F32)
    dwq = _dwconv_emul(qp, qdw_ref[...], TH_A)
    yfull = jnp.concatenate([yb_ref[0, 0], yc_ref[...], yb_ref[0, 1]],
                            axis=1)
    kp = jnp.dot(kw_ref[...], yfull.astype(BF),
                 preferred_element_type=F32)
    dwk = _dwconv_emul(kp, kdw_ref[...], TH_A)
    return dwq, dwk


def _pass_a_body(xc_ref, xb_ref, yc_ref, yb_ref, qw_ref, qdw_ref, kw_ref,
                 kdw_ref, w1_ref, b1_ref, w2_ref, b2_ref, temp_ref,
                 attn_ref, dk_ref, qsq_ref, ksq_ref, gs_ref):
    p = pl.program_id(0)
    i = pl.program_id(1)

    @pl.when((p == 0) & (i == 0))
    def _init():
        qsq_ref[...] = jnp.zeros_like(qsq_ref)
        ksq_ref[...] = jnp.zeros_like(ksq_ref)
        gs_ref[0, 0] = 0.0

    xc = xc_ref[...]
    dwq, dwk = _qk_tiles(xc, xb_ref, yc_ref, yb_ref, qw_ref, qdw_ref,
                         kw_ref, kdw_ref)

    @pl.when(p == 0)
    def _phase0():
        qsq_ref[...] += jnp.sum(dwq * dwq, axis=1, keepdims=True)
        ksq_ref[...] += jnp.sum(dwk * dwk, axis=1, keepdims=True)
        z1 = jnp.maximum(jnp.dot(w1_ref[...], xc.astype(BF),
                                 preferred_element_type=F32)
                         + b1_ref[...], 0.0)
        z2 = jax.nn.sigmoid(jnp.dot(w2_ref[...], z1.astype(BF),
                                    preferred_element_type=F32)
                            + b2_ref[...])
        z2 = jnp.where(jnp.isnan(z2), 0.0, z2)
        gs_ref[0, 0] += jnp.sum(z2)

        @pl.when(i == NSTEPS_A - 1)
        def _finalize_norms():
            qsq_ref[...] = jnp.maximum(jnp.sqrt(qsq_ref[...]), 1e-12)
            ksq_ref[...] = jnp.maximum(jnp.sqrt(ksq_ref[...]), 1e-12)
            gsv = gs_ref[0, 0]
            dkv = jnp.floor(CH * gsv / NPIX).astype(jnp.int32)
            dk_ref[0, 0] = jnp.maximum(dkv, 1)

    @pl.when(p == 1)
    def _phase1():
        @pl.when(i == 0)
        def _init_attn():
            attn_ref[...] = jnp.zeros_like(attn_ref)

        qn = (dwq / qsq_ref[...]).astype(BF)
        kn = (dwk / ksq_ref[...]).astype(BF)
        for h in range(HEADS):
            s_h = lax.dot_general(qn[h * CH:(h + 1) * CH],
                                  kn[h * CH:(h + 1) * CH],
                                  (((1,), (1,)), ((), ())),
                                  preferred_element_type=F32)
            attn_ref[h] += s_h

        @pl.when(i == NSTEPS_A - 1)
        def _finalize_attn():
            for h in range(HEADS):
                attn_ref[h] = attn_ref[h] * temp_ref[h, 0]


def _middle_body(a_ref, dk_ref, out_ref):
    a = a_ref[...].reshape(C, CH)
    dkv = dk_ref[0, 0]
    col = lax.broadcasted_iota(jnp.int32, (C, CH), 1)
    rank = jnp.zeros((C, CH), jnp.int32)
    for e in range(CH):
        ae = a[:, e:e + 1]
        pred = (ae > a) | ((ae == a) & (e < col))
        rank = rank + pred.astype(jnp.int32)
    mask = rank < dkv
    am = jnp.where(mask, a, NEG_INF)
    mx = jnp.max(am, axis=1, keepdims=True)
    ex = jnp.where(mask, jnp.exp(am - mx), 0.0)
    s = jnp.sum(ex, axis=1, keepdims=True)
    out_ref[...] = (ex / s).reshape(HEADS, CH, CH)


def _pass_c_body(yc_ref, yb_ref, vw_ref, vdw_ref, proj_ref, attn_ref,
                 out_ref):
    yfull = jnp.concatenate([yb_ref[0, 0], yc_ref[...], yb_ref[0, 1]],
                            axis=1)
    vp = jnp.dot(vw_ref[...], yfull.astype(BF), preferred_element_type=F32)
    dwv = _dwconv_emul_c(vp, vdw_ref[...])

    mm = jnp.concatenate(
        [jnp.dot(proj_ref[:, h * CH:(h + 1) * CH], attn_ref[h],
                 preferred_element_type=F32,
                 precision=lax.Precision.HIGHEST)
         for h in range(HEADS)], axis=1)
    out_ref[...] = jnp.dot(mm.astype(BF), dwv.astype(BF),
                           preferred_element_type=F32)


def _dwconv_emul_c(p_full, w9):
    return _dwconv_emul(p_full, w9, TH_C)


def _boundary_rows(x3, th):
    prev = x3[:, th - 1:HH - 1:th, :]
    nxt = x3[:, th:HH:th, :]
    zero = jnp.zeros((C, 1, WW), x3.dtype)
    prev_all = jnp.concatenate([zero, prev], axis=1).transpose(1, 0, 2)
    next_all = jnp.concatenate([nxt, zero], axis=1).transpose(1, 0, 2)
    return jnp.stack([prev_all, next_all], axis=1)


def kernel(x, y, q_w, q_dw_w, kv_w, kv_dw_w, proj_w, gate_w1, gate_b1,
           gate_w2, gate_b2, temperature):
    x3 = x.reshape(C, HH, WW)
    y3 = y.reshape(C, HH, WW)
    x2 = x.reshape(C, NPIX)
    y2 = y.reshape(C, NPIX)
    xb = _boundary_rows(x3, TH_A)
    yb = _boundary_rows(y3, TH_A)
    yb_c = _boundary_rows(y3, TH_C)
    k_w = kv_w[:C].astype(BF)
    v_w = kv_w[C:].astype(BF)
    qw_b = q_w.astype(BF)
    qdw = q_dw_w.reshape(C, 9)
    kdw = kv_dw_w[:C].reshape(C, 9)
    vdw = kv_dw_w[C:].reshape(C, 9)
    w1_b = gate_w1.astype(BF)
    w2_b = gate_w2.astype(BF)
    b1 = gate_b1.reshape(C // 2, 1)
    b2 = gate_b2.reshape(1, 1)
    temp = temperature.reshape(HEADS, 1)

    full = lambda shape: pl.BlockSpec(shape, lambda p, i: (0,) * len(shape))
    attn_s, dk = pl.pallas_call(
        _pass_a_body,
        grid=(2, NSTEPS_A),
        in_specs=[
            pl.BlockSpec((C, TH_A * WW), lambda p, i: (0, i)),
            pl.BlockSpec((1, 2, C, WW), lambda p, i: (i, 0, 0, 0)),
            pl.BlockSpec((C, TH_A * WW), lambda p, i: (0, i)),
            pl.BlockSpec((1, 2, C, WW), lambda p, i: (i, 0, 0, 0)),
            full((C, C)),
            full((C, 9)),
            full((C, C)),
            full((C, 9)),
            full((C // 2, C)),
            full((C // 2, 1)),
            full((1, C // 2)),
            full((1, 1)),
            pl.BlockSpec(memory_space=pltpu.SMEM),
        ],
        out_specs=[
            pl.BlockSpec((HEADS, CH, CH), lambda p, i: (0, 0, 0)),
            pl.BlockSpec(memory_space=pltpu.SMEM),
        ],
        out_shape=[
            jax.ShapeDtypeStruct((HEADS, CH, CH), jnp.float32),
            jax.ShapeDtypeStruct((1, 1), jnp.int32),
        ],
        scratch_shapes=[
            pltpu.VMEM((C, 1), jnp.float32),
            pltpu.VMEM((C, 1), jnp.float32),
            pltpu.SMEM((1, 1), jnp.float32),
        ],
        compiler_params=pltpu.CompilerParams(
            dimension_semantics=("arbitrary", "arbitrary")),
    )(x2, xb, y2, yb, qw_b, qdw, k_w, kdw, w1_b, b1, w2_b, b2, temp)

    attn = pl.pallas_call(
        _middle_body,
        in_specs=[
            pl.BlockSpec((HEADS, CH, CH), lambda: (0, 0, 0)),
            pl.BlockSpec(memory_space=pltpu.SMEM),
        ],
        out_specs=pl.BlockSpec((HEADS, CH, CH), lambda: (0, 0, 0)),
        out_shape=jax.ShapeDtypeStruct((HEADS, CH, CH), jnp.float32),
    )(attn_s, dk)

    out = pl.pallas_call(
        _pass_c_body,
        grid=(NSTEPS_C,),
        in_specs=[
            pl.BlockSpec((C, TH_C * WW), lambda i: (0, i)),
            pl.BlockSpec((1, 2, C, WW), lambda i: (i, 0, 0, 0)),
            pl.BlockSpec((C, C), lambda i: (0, 0)),
            pl.BlockSpec((C, 9), lambda i: (0, 0)),
            pl.BlockSpec((C, C), lambda i: (0, 0)),
            pl.BlockSpec((HEADS, CH, CH), lambda i: (0, 0, 0)),
        ],
        out_specs=pl.BlockSpec((C, TH_C * WW), lambda i: (0, i)),
        out_shape=jax.ShapeDtypeStruct((C, NPIX), jnp.float32),
        compiler_params=pltpu.CompilerParams(
            dimension_semantics=("arbitrary",)),
    )(y2, yb_c, v_w, vdw, proj_w, attn)

    return out.reshape(1, C, HH, WW)
